# Initial kernel scaffold; baseline (speedup 1.0000x reference)
import jax
import jax.numpy as jnp
from jax.experimental import pallas as pl


def kernel(texture, u, v):
    raise NotImplementedError("write your pallas kernel here")


# speedup vs baseline: 3.0080x; 3.0080x over previous
import functools

import jax
import jax.numpy as jnp
from jax import lax
from jax.experimental import pallas as pl
from jax.experimental.pallas import tpu as pltpu
from jax.experimental.pallas import tpu_sc as plsc

_L = 16

_BCAST_DNUMS = lax.GatherDimensionNumbers(
    offset_dims=(), collapsed_slice_dims=(0,), start_index_map=(0,)
)


def _lane_bcast(vec, sel):
    return lax.gather(
        vec,
        sel[:, None],
        _BCAST_DNUMS,
        slice_sizes=(1,),
        mode=lax.GatherScatterMode.PROMISE_IN_BOUNDS,
    )


def _make_sampler(h, w, ch, n_workers=32, chunk=128):
    n = h * w
    ppw = n // n_workers
    n_chunks = ppw // chunk
    groups = chunk // _L

    mesh = plsc.VectorSubcoreMesh(core_axis_name="c", subcore_axis_name="s")

    @functools.partial(
        pl.kernel,
        out_type=jax.ShapeDtypeStruct((n, ch), jnp.float32),
        mesh=mesh,
        scratch_types=[
            pltpu.VMEM((chunk,), jnp.float32),
            pltpu.VMEM((chunk,), jnp.float32),
            pltpu.VMEM((chunk,), jnp.float32),
            pltpu.VMEM((chunk,), jnp.float32),
            pltpu.VMEM((chunk,), jnp.int32),
            pltpu.VMEM((chunk,), jnp.int32),
            pltpu.VMEM((chunk,), jnp.int32),
            pltpu.VMEM((chunk,), jnp.int32),
            pltpu.VMEM((chunk, ch), jnp.float32),
            pltpu.VMEM((chunk, ch), jnp.float32),
            pltpu.VMEM((chunk, ch), jnp.float32),
            pltpu.VMEM((chunk, ch), jnp.float32),
            pltpu.VMEM((chunk, ch), jnp.float32),
            pltpu.SemaphoreType.DMA,
        ],
        compiler_params=pltpu.CompilerParams(use_tc_tiling_on_sc=False),
    )
    def sampler(tex_hbm, u_hbm, v_hbm, out_hbm,
                u_v, v_v, fx_v, fy_v,
                i00_v, i01_v, i10_v, i11_v,
                r00_v, r01_v, r10_v, r11_v, o_v, sem):
        cid = lax.axis_index("c")
        sid = lax.axis_index("s")
        wid = sid * 2 + cid
        base = wid * ppw

        def chunk_body(ci, carry):
            off = base + ci * chunk
            pltpu.sync_copy(u_hbm.at[pl.ds(off, chunk)], u_v)
            pltpu.sync_copy(v_hbm.at[pl.ds(off, chunk)], v_v)

            def idx_grp(g, c):
                s = g * _L
                uu = u_v[pl.ds(s, _L)]
                vv = v_v[pl.ds(s, _L)]
                x = uu * float(w) - 0.5
                y = vv * float(h) - 0.5
                xi = x.astype(jnp.int32)
                yi = y.astype(jnp.int32)
                x0 = jnp.where(xi.astype(jnp.float32) > x, xi - 1, xi)
                y0 = jnp.where(yi.astype(jnp.float32) > y, yi - 1, yi)
                fx_v[pl.ds(s, _L)] = x - x0.astype(jnp.float32)
                fy_v[pl.ds(s, _L)] = y - y0.astype(jnp.float32)
                x0 = jnp.where(x0 < 0, x0 + w, x0)
                y0 = jnp.where(y0 < 0, y0 + h, y0)
                x1 = x0 + 1
                x1 = jnp.where(x1 == w, 0, x1)
                y1 = y0 + 1
                y1 = jnp.where(y1 == h, 0, y1)
                r0 = y0 * w
                r1 = y1 * w
                i00_v[pl.ds(s, _L)] = r0 + x0
                i01_v[pl.ds(s, _L)] = r0 + x1
                i10_v[pl.ds(s, _L)] = r1 + x0
                i11_v[pl.ds(s, _L)] = r1 + x1
                return c

            lax.fori_loop(0, groups, idx_grp, 0)

            c00 = pltpu.async_copy(tex_hbm.at[i00_v], r00_v, sem)
            c01 = pltpu.async_copy(tex_hbm.at[i01_v], r01_v, sem)
            c10 = pltpu.async_copy(tex_hbm.at[i10_v], r10_v, sem)
            c11 = pltpu.async_copy(tex_hbm.at[i11_v], r11_v, sem)
            c00.wait()
            c01.wait()
            c10.wait()
            c11.wait()

            def blend_grp(g, c):
                s = g * _L
                fx16 = fx_v[pl.ds(s, _L)]
                fy16 = fy_v[pl.ds(s, _L)]
                for lp in range(_L):
                    p = s + lp
                    sel = jnp.full((_L,), lp, jnp.int32)
                    fxp = _lane_bcast(fx16, sel)
                    fyp = _lane_bcast(fy16, sel)
                    gxp = 1.0 - fxp
                    gyp = 1.0 - fyp
                    w00 = gxp * gyp
                    w01 = fxp * gyp
                    w10 = gxp * fyp
                    w11 = fxp * fyp
                    for half in range(ch // _L):
                        cs = half * _L
                        v00 = r00_v[p, pl.ds(cs, _L)]
                        v01 = r01_v[p, pl.ds(cs, _L)]
                        v10 = r10_v[p, pl.ds(cs, _L)]
                        v11 = r11_v[p, pl.ds(cs, _L)]
                        o_v[p, pl.ds(cs, _L)] = (
                            v00 * w00 + v01 * w01 + v10 * w10 + v11 * w11
                        )
                return c

            lax.fori_loop(0, groups, blend_grp, 0)

            pltpu.sync_copy(o_v, out_hbm.at[pl.ds(off, chunk)])
            return carry

        lax.fori_loop(0, n_chunks, chunk_body, 0)

    return sampler


def kernel(texture, u, v):
    h, w, ch = texture.shape
    n = h * w
    sampler = _make_sampler(h, w, ch)
    out = sampler(
        texture.reshape(n, ch),
        u.reshape(n),
        v.reshape(n),
    )
    return out.reshape(h, w, ch)
